# baseline (speedup 1.0000x reference)
import functools

import jax
import jax.numpy as jnp
from jax import lax
from jax.experimental import pallas as pl
from jax.experimental.pallas import tpu as pltpu
from jax.experimental.pallas import tpu_sc as plsc

_B, _D, _P, _C, _T = 1024, 1024, 500, 1000, 20000
_NC, _NS, _L = 2, 16, 16
_NW = _NC * _NS
_ROWS = _B // _NW
_TB = 640
_NF = _T // _TB
_RB = _T - _NF * _TB
_NV = _TB // _L
_RV = _RB // _L
_UNROLL = 10


def _mm_body(feat_ref, wt_ref, b_ref, out_ref):
    out_ref[...] = (
        jnp.dot(feat_ref[...], wt_ref[...], preferred_element_type=jnp.float32)
        + b_ref[...]
    )


def _linear(feat, W, b):
    return pl.pallas_call(
        _mm_body,
        out_shape=jax.ShapeDtypeStruct((_B, _P), jnp.float32),
    )(feat, W.T, b.reshape(1, _P))


def _gather_body(ps_hbm, p_hbm, po_hbm, ss_hbm, sp_hbm, so_hbm, out_hbm,
                 s_tile, p_tile, o_tile,
                 idx_s0, idx_p0, idx_o0, idx_s1, idx_p1, idx_o1,
                 out0, out1, sem_i0, sem_i1, sem_o0, sem_o1):
    wid = lax.axis_index("s") * _NC + lax.axis_index("c")
    rbase = wid * _ROWS
    slots = ((idx_s0, idx_p0, idx_o0, out0, sem_i0, sem_o0),
             (idx_s1, idx_p1, idx_o1, out1, sem_i1, sem_o1))

    def idx_start(c, slot):
        idx_s, idx_p, idx_o, _, sem_i, _ = slot
        tbase = c * _TB
        pltpu.async_copy(ss_hbm.at[pl.ds(tbase, _TB)], idx_s, sem_i)
        pltpu.async_copy(sp_hbm.at[pl.ds(tbase, _TB)], idx_p, sem_i)
        pltpu.async_copy(so_hbm.at[pl.ds(tbase, _TB)], idx_o, sem_i)

    def idx_wait(slot):
        idx_s, idx_p, idx_o, _, sem_i, _ = slot
        pltpu.make_async_copy(ss_hbm.at[pl.ds(0, _TB)], idx_s, sem_i).wait()
        pltpu.make_async_copy(sp_hbm.at[pl.ds(0, _TB)], idx_p, sem_i).wait()
        pltpu.make_async_copy(so_hbm.at[pl.ds(0, _TB)], idx_o, sem_i).wait()

    def compute(slot, unroll):
        idx_s, idx_p, idx_o, obuf, _, _ = slot

        @plsc.parallel_loop(0, _NV, unroll=unroll)
        def _vec(v):
            off = v * _L
            cs = idx_s[pl.ds(off, _L)]
            cp = idx_p[pl.ds(off, _L)]
            co = idx_o[pl.ds(off, _L)]
            for r in range(_ROWS):
                rv = jnp.full((_L,), r, jnp.int32)
                sv = plsc.load_gather(s_tile, [rv, cs])
                pv = plsc.load_gather(p_tile, [rv, cp])
                ov = plsc.load_gather(o_tile, [rv, co])
                obuf[pl.ds(r * _TB + off, _L)] = sv * pv * ov

    def out_start(c, slot):
        _, _, _, obuf, _, sem_o = slot
        tbase = c * _TB
        for r in range(_ROWS):
            pltpu.async_copy(
                obuf.at[pl.ds(r * _TB, _TB)],
                out_hbm.at[pl.ds((rbase + r) * _T + tbase, _TB)], sem_o)

    def out_wait(slot):
        _, _, _, obuf, _, sem_o = slot
        pltpu.make_async_copy(out_hbm.at[pl.ds(0, _ROWS * _TB)], obuf,
                              sem_o).wait()

    pltpu.sync_copy(ps_hbm.at[pl.ds(rbase, _ROWS), :], s_tile)
    pltpu.sync_copy(p_hbm.at[pl.ds(rbase, _ROWS), :], p_tile)
    pltpu.sync_copy(po_hbm.at[pl.ds(rbase, _ROWS), :], o_tile)

    idx_start(0, slots[0])
    idx_start(1, slots[1])

    @pl.loop(0, _NF - 1, step=2)
    def _chunk(ci):
        for k in range(2):
            c = ci + k

            @pl.when(c >= 2)
            def _():
                out_wait(slots[k])

            idx_wait(slots[k])
            compute(slots[k], _UNROLL)
            out_start(c, slots[k])

            @pl.when(c + 2 < _NF)
            def _():
                idx_start(c + 2, slots[k])

    out_wait(slots[0])
    idx_wait(slots[0])
    compute(slots[0], _UNROLL)
    out_start(_NF - 1, slots[0])

    out_wait(slots[1])
    rem = _NF * _TB
    pltpu.sync_copy(ss_hbm.at[pl.ds(rem, _RB)], idx_s1.at[pl.ds(0, _RB)])
    pltpu.sync_copy(sp_hbm.at[pl.ds(rem, _RB)], idx_p1.at[pl.ds(0, _RB)])
    pltpu.sync_copy(so_hbm.at[pl.ds(rem, _RB)], idx_o1.at[pl.ds(0, _RB)])

    @plsc.parallel_loop(0, _RV, unroll=5)
    def _tvec(v):
        off = v * _L
        cs = idx_s1[pl.ds(off, _L)]
        cp = idx_p1[pl.ds(off, _L)]
        co = idx_o1[pl.ds(off, _L)]
        for r in range(_ROWS):
            rv = jnp.full((_L,), r, jnp.int32)
            sv = plsc.load_gather(s_tile, [rv, cs])
            pv = plsc.load_gather(p_tile, [rv, cp])
            ov = plsc.load_gather(o_tile, [rv, co])
            out1[pl.ds(r * _RB + off, _L)] = sv * pv * ov

    for r in range(_ROWS):
        pltpu.async_copy(out1.at[pl.ds(r * _RB, _RB)],
                         out_hbm.at[pl.ds((rbase + r) * _T + rem, _RB)],
                         sem_o1)
    pltpu.make_async_copy(out_hbm.at[pl.ds(0, _ROWS * _RB)],
                          out1.at[pl.ds(0, _ROWS * _RB)], sem_o1).wait()
    out_wait(slots[0])


_gather = functools.partial(
    pl.kernel,
    out_type=jax.ShapeDtypeStruct((_B * _T,), jnp.float32),
    mesh=plsc.VectorSubcoreMesh(
        core_axis_name="c", subcore_axis_name="s",
        num_cores=_NC, num_subcores=_NS),
    compiler_params=pltpu.CompilerParams(needs_layout_passes=False),
    scratch_types=[
        pltpu.VMEM((_ROWS, _C), jnp.float32),
        pltpu.VMEM((_ROWS, _P), jnp.float32),
        pltpu.VMEM((_ROWS, _C), jnp.float32),
        pltpu.VMEM((_TB,), jnp.int32),
        pltpu.VMEM((_TB,), jnp.int32),
        pltpu.VMEM((_TB,), jnp.int32),
        pltpu.VMEM((_TB,), jnp.int32),
        pltpu.VMEM((_TB,), jnp.int32),
        pltpu.VMEM((_TB,), jnp.int32),
        pltpu.VMEM((_ROWS * _TB,), jnp.float32),
        pltpu.VMEM((_ROWS * _TB,), jnp.float32),
        pltpu.SemaphoreType.DMA,
        pltpu.SemaphoreType.DMA,
        pltpu.SemaphoreType.DMA,
        pltpu.SemaphoreType.DMA,
    ],
)(_gather_body)


def kernel(feat, prob_s, prob_o, W, b, sel_s, sel_p, sel_o):
    p = _linear(feat, W, b)
    flat = _gather(prob_s, p, prob_o, sel_s, sel_p, sel_o)
    return jnp.reshape(flat, (_B, _T))


# speedup vs baseline: 1.0975x; 1.0975x over previous
import functools

import jax
import jax.numpy as jnp
from jax import lax
from jax.experimental import pallas as pl
from jax.experimental.pallas import tpu as pltpu
from jax.experimental.pallas import tpu_sc as plsc

_B, _D, _P, _C, _T = 1024, 1024, 500, 1000, 20000
_NC, _NS, _L = 2, 16, 16
_NW = _NC * _NS
_ROWS = _B // _NW
_TB = 640
_NF = _T // _TB
_RB = _T - _NF * _TB
_NV = _TB // _L
_RV = _RB // _L
_UNROLL = 10


def _mm_body(feat_ref, wt_ref, b_ref, out_ref):
    out_ref[...] = (
        jnp.dot(feat_ref[...], wt_ref[...], preferred_element_type=jnp.float32)
        + b_ref[...]
    )


def _linear(feat, W, b):
    return pl.pallas_call(
        _mm_body,
        out_shape=jax.ShapeDtypeStruct((_B, _P), jnp.float32),
    )(feat, W.T, b.reshape(1, _P))


def _gather_body(ps_hbm, p_hbm, po_hbm, ss_hbm, sp_hbm, so_hbm, out_hbm,
                 s_tile, p_tile, o_tile,
                 idx_s0, idx_p0, idx_o0, idx_s1, idx_p1, idx_o1,
                 out0, out1, sem_i0, sem_i1, sem_o0, sem_o1):
    wid = lax.axis_index("s") * _NC + lax.axis_index("c")
    rbase = wid * _ROWS
    slots = ((idx_s0, idx_p0, idx_o0, out0, sem_i0, sem_o0),
             (idx_s1, idx_p1, idx_o1, out1, sem_i1, sem_o1))

    def idx_start(c, slot, width):
        idx_s, idx_p, idx_o, _, sem_i, _ = slot
        tbase = c * _TB
        pltpu.async_copy(ss_hbm.at[pl.ds(tbase, width)],
                         idx_s.at[pl.ds(0, width)], sem_i)
        pltpu.async_copy(sp_hbm.at[pl.ds(tbase, width)],
                         idx_p.at[pl.ds(0, width)], sem_i)
        pltpu.async_copy(so_hbm.at[pl.ds(tbase, width)],
                         idx_o.at[pl.ds(0, width)], sem_i)

    def idx_wait(slot, width):
        idx_s, idx_p, idx_o, _, sem_i, _ = slot
        pltpu.make_async_copy(ss_hbm.at[pl.ds(0, width)],
                              idx_s.at[pl.ds(0, width)], sem_i).wait()
        pltpu.make_async_copy(sp_hbm.at[pl.ds(0, width)],
                              idx_p.at[pl.ds(0, width)], sem_i).wait()
        pltpu.make_async_copy(so_hbm.at[pl.ds(0, width)],
                              idx_o.at[pl.ds(0, width)], sem_i).wait()

    def compute(slot, nv, unroll):
        idx_s, idx_p, idx_o, obuf, _, _ = slot

        @plsc.parallel_loop(0, nv, unroll=unroll)
        def _vec(v):
            off = v * _L
            cs = idx_s[pl.ds(off, _L)]
            cp = idx_p[pl.ds(off, _L)]
            co = idx_o[pl.ds(off, _L)]
            for r in range(_ROWS):
                rv = jnp.full((_L,), r, jnp.int32)
                sv = plsc.load_gather(s_tile, [rv, cs])
                pv = plsc.load_gather(p_tile, [rv, cp])
                ov = plsc.load_gather(o_tile, [rv, co])
                obuf[r, pl.ds(off, _L)] = sv * pv * ov

    def out_start(c, slot, width):
        _, _, _, obuf, _, sem_o = slot
        pltpu.async_copy(
            obuf.at[:, pl.ds(0, width)],
            out_hbm.at[pl.ds(rbase, _ROWS), pl.ds(c * _TB, width)], sem_o)

    def out_wait(slot, width):
        _, _, _, obuf, _, sem_o = slot
        pltpu.make_async_copy(
            obuf.at[:, pl.ds(0, width)],
            out_hbm.at[pl.ds(rbase, _ROWS), pl.ds(0, width)], sem_o).wait()

    pltpu.sync_copy(ps_hbm.at[pl.ds(rbase, _ROWS), :], s_tile)
    pltpu.sync_copy(p_hbm.at[pl.ds(rbase, _ROWS), :], p_tile)
    pltpu.sync_copy(po_hbm.at[pl.ds(rbase, _ROWS), :], o_tile)

    idx_start(0, slots[0], _TB)
    idx_start(1, slots[1], _TB)

    @pl.loop(0, _NF - 1, step=2)
    def _chunk(ci):
        for k in range(2):
            c = ci + k

            @pl.when(c >= 2)
            def _():
                out_wait(slots[k], _TB)

            idx_wait(slots[k], _TB)
            compute(slots[k], _NV, _UNROLL)
            out_start(c, slots[k], _TB)

            @pl.when(c + 2 < _NF)
            def _():
                idx_start(c + 2, slots[k], _TB)

    out_wait(slots[0], _TB)
    idx_wait(slots[0], _TB)
    compute(slots[0], _NV, _UNROLL)
    out_start(_NF - 1, slots[0], _TB)

    out_wait(slots[1], _TB)
    rem = _NF * _TB
    pltpu.sync_copy(ss_hbm.at[pl.ds(rem, _RB)], idx_s1.at[pl.ds(0, _RB)])
    pltpu.sync_copy(sp_hbm.at[pl.ds(rem, _RB)], idx_p1.at[pl.ds(0, _RB)])
    pltpu.sync_copy(so_hbm.at[pl.ds(rem, _RB)], idx_o1.at[pl.ds(0, _RB)])
    compute(slots[1], _RV, 5)
    pltpu.sync_copy(out1.at[:, pl.ds(0, _RB)],
                    out_hbm.at[pl.ds(rbase, _ROWS), pl.ds(rem, _RB)])
    out_wait(slots[0], _TB)


_gather = functools.partial(
    pl.kernel,
    out_type=jax.ShapeDtypeStruct((_B, _T), jnp.float32),
    mesh=plsc.VectorSubcoreMesh(
        core_axis_name="c", subcore_axis_name="s",
        num_cores=_NC, num_subcores=_NS),
    compiler_params=pltpu.CompilerParams(
        use_tc_tiling_on_sc=False, needs_layout_passes=False),
    scratch_types=[
        pltpu.VMEM((_ROWS, _C), jnp.float32),
        pltpu.VMEM((_ROWS, _P), jnp.float32),
        pltpu.VMEM((_ROWS, _C), jnp.float32),
        pltpu.VMEM((_TB,), jnp.int32),
        pltpu.VMEM((_TB,), jnp.int32),
        pltpu.VMEM((_TB,), jnp.int32),
        pltpu.VMEM((_TB,), jnp.int32),
        pltpu.VMEM((_TB,), jnp.int32),
        pltpu.VMEM((_TB,), jnp.int32),
        pltpu.VMEM((_ROWS, _TB), jnp.float32),
        pltpu.VMEM((_ROWS, _TB), jnp.float32),
        pltpu.SemaphoreType.DMA,
        pltpu.SemaphoreType.DMA,
        pltpu.SemaphoreType.DMA,
        pltpu.SemaphoreType.DMA,
    ],
)(_gather_body)


def kernel(feat, prob_s, prob_o, W, b, sel_s, sel_p, sel_o):
    p = _linear(feat, W, b)
    return _gather(prob_s, p, prob_o, sel_s, sel_p, sel_o)


# speedup vs baseline: 1.6315x; 1.4865x over previous
import functools

import jax
import jax.numpy as jnp
from jax import lax
from jax.experimental import pallas as pl
from jax.experimental.pallas import tpu as pltpu
from jax.experimental.pallas import tpu_sc as plsc

_B, _D, _P, _C, _T = 1024, 1024, 500, 1000, 20000
_TP = 20096
_NC, _NS, _L = 2, 16, 16
_NW = _NC * _NS
_ROWS = _B // _NW
_TB = 512
_NF = 39
_LB = 128
_LVALID = _T - _NF * _TB
_NV = _TB // _L
_LV = _LB // _L
_UNROLL = 4


def _mm_body(feat_ref, wt_ref, b_ref, out_ref):
    out_ref[...] = (
        jnp.dot(feat_ref[...], wt_ref[...], preferred_element_type=jnp.float32)
        + b_ref[...]
    )


def _linear(feat, W, b):
    return pl.pallas_call(
        _mm_body,
        out_shape=jax.ShapeDtypeStruct((_B, _P), jnp.float32),
    )(feat, W.T, b.reshape(1, _P))


def _gather_body(ps_hbm, p_hbm, po_hbm, ss_hbm, sp_hbm, so_hbm, out_hbm,
                 s_tile, p_tile, o_tile,
                 idx_s0, idx_p0, idx_o0, idx_s1, idx_p1, idx_o1,
                 out0, out1, sem_i0, sem_i1, sem_o0, sem_o1):
    wid = lax.axis_index("s") * _NC + lax.axis_index("c")
    rbase = wid * _ROWS
    slots = ((idx_s0, idx_p0, idx_o0, out0, sem_i0, sem_o0),
             (idx_s1, idx_p1, idx_o1, out1, sem_i1, sem_o1))

    def idx_start(c, slot):
        idx_s, idx_p, idx_o, _, sem_i, _ = slot
        tbase = c * _TB
        pltpu.async_copy(ss_hbm.at[pl.ds(tbase, _TB)], idx_s, sem_i)
        pltpu.async_copy(sp_hbm.at[pl.ds(tbase, _TB)], idx_p, sem_i)
        pltpu.async_copy(so_hbm.at[pl.ds(tbase, _TB)], idx_o, sem_i)

    def idx_wait(slot):
        idx_s, idx_p, idx_o, _, sem_i, _ = slot
        pltpu.make_async_copy(ss_hbm.at[pl.ds(0, _TB)], idx_s, sem_i).wait()
        pltpu.make_async_copy(sp_hbm.at[pl.ds(0, _TB)], idx_p, sem_i).wait()
        pltpu.make_async_copy(so_hbm.at[pl.ds(0, _TB)], idx_o, sem_i).wait()

    def compute(slot, nv):
        idx_s, idx_p, idx_o, obuf, _, _ = slot

        @plsc.parallel_loop(0, nv, unroll=_UNROLL)
        def _vec(v):
            off = v * _L
            cs = idx_s[pl.ds(off, _L)]
            cp = idx_p[pl.ds(off, _L)]
            co = idx_o[pl.ds(off, _L)]
            for r in range(_ROWS):
                rv = jnp.full((_L,), r, jnp.int32)
                sv = plsc.load_gather(s_tile, [rv, cs])
                pv = plsc.load_gather(p_tile, [rv, cp])
                ov = plsc.load_gather(o_tile, [rv, co])
                obuf[r, pl.ds(off, _L)] = sv * pv * ov

    def out_start(c, slot, width):
        _, _, _, obuf, _, sem_o = slot
        pltpu.async_copy(
            obuf.at[:, pl.ds(0, width)],
            out_hbm.at[pl.ds(rbase, _ROWS), pl.ds(c * _TB, width)], sem_o)

    def out_wait(slot, width):
        _, _, _, obuf, _, sem_o = slot
        pltpu.make_async_copy(
            obuf.at[:, pl.ds(0, width)],
            out_hbm.at[pl.ds(rbase, _ROWS), pl.ds(0, width)], sem_o).wait()

    pltpu.sync_copy(ps_hbm.at[pl.ds(rbase, _ROWS), :], s_tile)
    pltpu.sync_copy(p_hbm.at[pl.ds(rbase, _ROWS), :], p_tile)
    pltpu.sync_copy(po_hbm.at[pl.ds(rbase, _ROWS), :], o_tile)

    idx_start(0, slots[0])
    idx_start(1, slots[1])
    for k in range(2):
        idx_wait(slots[k])
        compute(slots[k], _NV)
        out_start(k, slots[k], _TB)
        idx_start(k + 2, slots[k])

    @pl.loop(2, 36, step=2)
    def _chunk(ci):
        for k in range(2):
            c = ci + k
            out_wait(slots[k], _TB)
            idx_wait(slots[k])
            compute(slots[k], _NV)
            out_start(c, slots[k], _TB)
            idx_start(c + 2, slots[k])

    for k in range(2):
        c = 36 + k
        out_wait(slots[k], _TB)
        idx_wait(slots[k])
        compute(slots[k], _NV)
        out_start(c, slots[k], _TB)
        if c + 2 < _NF:
            idx_start(c + 2, slots[k])

    out_wait(slots[0], _TB)
    idx_wait(slots[0])
    compute(slots[0], _NV)
    out_start(38, slots[0], _TB)

    out_wait(slots[1], _TB)
    rem = _NF * _TB
    pltpu.sync_copy(ss_hbm.at[pl.ds(rem, _LVALID)],
                    idx_s1.at[pl.ds(0, _LVALID)])
    pltpu.sync_copy(sp_hbm.at[pl.ds(rem, _LVALID)],
                    idx_p1.at[pl.ds(0, _LVALID)])
    pltpu.sync_copy(so_hbm.at[pl.ds(rem, _LVALID)],
                    idx_o1.at[pl.ds(0, _LVALID)])
    z = jnp.zeros((_L,), jnp.int32)
    for j in range(_LVALID, _LB, _L):
        idx_s1[pl.ds(j, _L)] = z
        idx_p1[pl.ds(j, _L)] = z
        idx_o1[pl.ds(j, _L)] = z
    compute(slots[1], _LV)
    out_start(39, slots[1], _LB)
    out_wait(slots[0], _TB)
    out_wait(slots[1], _LB)


_gather = functools.partial(
    pl.kernel,
    out_type=jax.ShapeDtypeStruct((_B, _TP), jnp.float32),
    mesh=plsc.VectorSubcoreMesh(
        core_axis_name="c", subcore_axis_name="s",
        num_cores=_NC, num_subcores=_NS),
    compiler_params=pltpu.CompilerParams(needs_layout_passes=False),
    scratch_types=[
        pltpu.VMEM((_ROWS, _C), jnp.float32),
        pltpu.VMEM((_ROWS, _P), jnp.float32),
        pltpu.VMEM((_ROWS, _C), jnp.float32),
        pltpu.VMEM((_TB,), jnp.int32),
        pltpu.VMEM((_TB,), jnp.int32),
        pltpu.VMEM((_TB,), jnp.int32),
        pltpu.VMEM((_TB,), jnp.int32),
        pltpu.VMEM((_TB,), jnp.int32),
        pltpu.VMEM((_TB,), jnp.int32),
        pltpu.VMEM((_ROWS, _TB), jnp.float32),
        pltpu.VMEM((_ROWS, _TB), jnp.float32),
        pltpu.SemaphoreType.DMA,
        pltpu.SemaphoreType.DMA,
        pltpu.SemaphoreType.DMA,
        pltpu.SemaphoreType.DMA,
    ],
)(_gather_body)


def kernel(feat, prob_s, prob_o, W, b, sel_s, sel_p, sel_o):
    p = _linear(feat, W, b)
    padded = _gather(prob_s, p, prob_o, sel_s, sel_p, sel_o)
    return lax.slice(padded, (0, 0), (_B, _T))
